# baseline (speedup 1.0000x reference)
import functools

import jax
import jax.numpy as jnp
from jax import lax
from jax.experimental import pallas as pl
from jax.experimental.pallas import tpu as pltpu
from jax.experimental.pallas import tpu_sc as plsc

B, S, D = 4096, 200, 64
DQ = D // 4
N_TOK = B * S
N_ROWS = 24 * 7 * 12
DH = D // 2

NC, NS = 2, 16
NW = NC * NS
N_QUAD = N_TOK // 4
QB = B // 4
Q_PER_W = N_QUAD // NW
HALF = 320
N_CHUNKS = Q_PER_W // HALF

BB = 128


def _table_body(hour_ref, week_ref, month_ref, season_ref, w_ref, b_ref, ft_ref):
    w = w_ref[...]
    ht = jnp.dot(hour_ref[...], w[:, 0:DQ].T, preferred_element_type=jnp.float32)
    wt = jnp.dot(week_ref[...], w[:, DQ:2 * DQ].T, preferred_element_type=jnp.float32)
    mt = jnp.dot(month_ref[...], w[:, 2 * DQ:3 * DQ].T, preferred_element_type=jnp.float32)
    st = jnp.dot(season_ref[...], w[:, 3 * DQ:4 * DQ].T, preferred_element_type=jnp.float32)
    s_oh = (lax.broadcasted_iota(jnp.int32, (12, 4), 0) // 3
            == lax.broadcasted_iota(jnp.int32, (12, 4), 1)).astype(jnp.float32)
    mt2 = mt + jnp.dot(s_oh, st, preferred_element_type=jnp.float32) + b_ref[...][None, :]
    c_h = lax.broadcasted_iota(jnp.int32, (N_ROWS, 24), 0) // 84
    oh_h = (c_h == lax.broadcasted_iota(jnp.int32, (N_ROWS, 24), 1)).astype(jnp.float32)
    c_w = (lax.broadcasted_iota(jnp.int32, (N_ROWS, 7), 0) // 12) % 7
    oh_w = (c_w == lax.broadcasted_iota(jnp.int32, (N_ROWS, 7), 1)).astype(jnp.float32)
    c_m = lax.broadcasted_iota(jnp.int32, (N_ROWS, 12), 0) % 12
    oh_m = (c_m == lax.broadcasted_iota(jnp.int32, (N_ROWS, 12), 1)).astype(jnp.float32)
    ft = (jnp.dot(oh_h, ht, preferred_element_type=jnp.float32)
          + jnp.dot(oh_w, wt, preferred_element_type=jnp.float32)
          + jnp.dot(oh_m, mt2, preferred_element_type=jnp.float32))
    lo = lax.bitcast_convert_type(ft[:, :DH].astype(jnp.bfloat16), jnp.uint16)
    hi = lax.bitcast_convert_type(ft[:, DH:].astype(jnp.bfloat16), jnp.uint16)
    ft_ref[...] = (lo.astype(jnp.int32)
                   | (hi.astype(jnp.int32) << 16))


def _build_table(hour_table, weekday_table, month_table, season_table, w, b):
    return pl.pallas_call(
        _table_body,
        out_shape=jax.ShapeDtypeStruct((N_ROWS, DH), jnp.int32),
    )(hour_table, weekday_table, month_table, season_table, w, b)


def _idx_body(ts_ref, idx_ref):
    t = ts_ref[...]
    h = (t // 60) % 24
    wd = (t // 1440) % 7
    m = (t // 43200) % 12
    idx_ref[...] = h * 84 + wd * 12 + m


def _build_idx(timestamps):
    blk = 512
    return pl.pallas_call(
        _idx_body,
        grid=(B // blk,),
        in_specs=[pl.BlockSpec((blk, S), lambda i: (i, 0))],
        out_specs=pl.BlockSpec((blk, S), lambda i: (i, 0)),
        out_shape=jax.ShapeDtypeStruct((B, S), jnp.int32),
    )(timestamps)


def _sc_gather_body(ft_hbm, i0_hbm, i1_hbm, i2_hbm, i3_hbm, out_hbm,
                    x0, x1, x2, x3,
                    r00, r01, r02, r03, r10, r11, r12, r13,
                    gsem0, gsem1, osem0, osem1):
    wid = lax.axis_index("s") * NC + lax.axis_index("c")
    q0 = wid * Q_PER_W
    idx_all = (x0, x1, x2, x3)
    rows = ((r00, r01, r02, r03), (r10, r11, r12, r13))
    gsem = (gsem0, gsem1)
    osem = (osem0, osem1)

    pltpu.sync_copy(i0_hbm.at[pl.ds(q0, Q_PER_W)], x0)
    pltpu.sync_copy(i1_hbm.at[pl.ds(q0, Q_PER_W)], x1)
    pltpu.sync_copy(i2_hbm.at[pl.ds(q0, Q_PER_W)], x2)
    pltpu.sync_copy(i3_hbm.at[pl.ds(q0, Q_PER_W)], x3)

    def issue_gathers(i, k):
        off = i * HALF
        for q in range(4):
            pltpu.async_copy(ft_hbm.at[idx_all[q].at[pl.ds(off, HALF)]],
                             rows[k][q], gsem[k])

    def wait_gathers(k):
        for q in range(4):
            pltpu.make_async_copy(ft_hbm.at[pl.ds(0, HALF)], rows[k][q],
                                  gsem[k]).wait()

    def issue_writes(i, k):
        base = q0 + i * HALF
        for q in range(4):
            pltpu.async_copy(rows[k][q],
                             out_hbm.at[pl.ds(base, HALF), pl.ds(q * DH, DH)],
                             osem[k])

    def wait_writes(k):
        for q in range(4):
            pltpu.make_async_copy(rows[k][q],
                                  out_hbm.at[pl.ds(0, HALF), pl.ds(q * DH, DH)],
                                  osem[k]).wait()

    issue_gathers(0, 0)
    issue_gathers(1, 1)
    wait_gathers(0)
    issue_writes(0, 0)

    def steady(j, _):
        for k in (0, 1):
            i = 2 + 2 * j + k
            wait_writes(k)
            wait_gathers(1 - k)
            issue_writes(i - 1, 1 - k)
            issue_gathers(i, k)
        return 0

    lax.fori_loop(0, (N_CHUNKS - 2) // 2, steady, 0)

    wait_gathers(1)
    issue_writes(N_CHUNKS - 1, 1)
    wait_writes(0)
    wait_writes(1)


@functools.cache
def _sc_gather():
    return functools.partial(
        pl.kernel,
        mesh=plsc.VectorSubcoreMesh(core_axis_name="c", subcore_axis_name="s"),
        out_type=jax.ShapeDtypeStruct((N_QUAD, 4 * DH), jnp.int32),
        scratch_types=(
            [pltpu.VMEM((Q_PER_W,), jnp.int32)] * 4
            + [pltpu.VMEM((HALF, DH), jnp.int32)] * 8
            + [pltpu.SemaphoreType.DMA] * 4
        ),
        compiler_params=pltpu.CompilerParams(use_tc_tiling_on_sc=False),
    )(_sc_gather_body)


def _transpose_body(x_ref, o_ref):
    i = pl.program_id(0)
    x = x_ref[...]
    q = i // (QB // BB)
    a = x[:, :, 0:DH]
    for qq in range(1, 4):
        a = jnp.where(q == qq, x[:, :, qq * DH:(qq + 1) * DH], a)
    a_t = jnp.transpose(a, (0, 2, 1))
    lo = lax.bitcast_convert_type(a_t << 16, jnp.float32)
    hi = lax.bitcast_convert_type(a_t & jnp.int32(-65536), jnp.float32)
    o_ref[...] = jnp.concatenate([lo, hi], axis=1)


def _transpose(x3):
    return pl.pallas_call(
        _transpose_body,
        grid=(B // BB,),
        in_specs=[pl.BlockSpec((S, BB, 128), lambda i: (0, i % (QB // BB), 0))],
        out_specs=pl.BlockSpec((S, D, BB), lambda i: (0, 0, i)),
        out_shape=jax.ShapeDtypeStruct((S, D, B), jnp.float32),
    )(x3)


def kernel(timestamps, hour_table, weekday_table, month_table, season_table, W, b):
    ft = _build_table(hour_table, weekday_table, month_table, season_table, W, b)
    idx = _build_idx(timestamps)
    iq = [idx[q * QB:(q + 1) * QB, :].T.reshape(N_QUAD) for q in range(4)]
    quads = _sc_gather()(ft, iq[0], iq[1], iq[2], iq[3])
    ot = _transpose(quads.reshape(S, QB, 128))
    return jnp.transpose(ot, (2, 0, 1))


# speedup vs baseline: 1.4536x; 1.4536x over previous
import functools

import jax
import jax.numpy as jnp
from jax import lax
from jax.experimental import pallas as pl
from jax.experimental.pallas import tpu as pltpu
from jax.experimental.pallas import tpu_sc as plsc

B, S, D = 4096, 200, 64
DQ = D // 4
N_TOK = B * S
N_ROWS = 24 * 7 * 12

NC, NS = 2, 16
NW = NC * NS
N_PAIR = N_TOK // 2
PAIR_PER_W = N_PAIR // NW
HALF = 320
N_CHUNKS = PAIR_PER_W // HALF

HB = B // 2
BB = 128


def _table_body(hour_ref, week_ref, month_ref, season_ref, w_ref, b_ref, ft_ref):
    w = w_ref[...]
    ht = jnp.dot(hour_ref[...], w[:, 0:DQ].T, preferred_element_type=jnp.float32)
    wt = jnp.dot(week_ref[...], w[:, DQ:2 * DQ].T, preferred_element_type=jnp.float32)
    mt = jnp.dot(month_ref[...], w[:, 2 * DQ:3 * DQ].T, preferred_element_type=jnp.float32)
    st = jnp.dot(season_ref[...], w[:, 3 * DQ:4 * DQ].T, preferred_element_type=jnp.float32)
    s_oh = (lax.broadcasted_iota(jnp.int32, (12, 4), 0) // 3
            == lax.broadcasted_iota(jnp.int32, (12, 4), 1)).astype(jnp.float32)
    mt2 = mt + jnp.dot(s_oh, st, preferred_element_type=jnp.float32) + b_ref[...][None, :]
    c_h = lax.broadcasted_iota(jnp.int32, (N_ROWS, 24), 0) // 84
    oh_h = (c_h == lax.broadcasted_iota(jnp.int32, (N_ROWS, 24), 1)).astype(jnp.float32)
    c_w = (lax.broadcasted_iota(jnp.int32, (N_ROWS, 7), 0) // 12) % 7
    oh_w = (c_w == lax.broadcasted_iota(jnp.int32, (N_ROWS, 7), 1)).astype(jnp.float32)
    c_m = lax.broadcasted_iota(jnp.int32, (N_ROWS, 12), 0) % 12
    oh_m = (c_m == lax.broadcasted_iota(jnp.int32, (N_ROWS, 12), 1)).astype(jnp.float32)
    ft_ref[...] = (jnp.dot(oh_h, ht, preferred_element_type=jnp.float32)
                   + jnp.dot(oh_w, wt, preferred_element_type=jnp.float32)
                   + jnp.dot(oh_m, mt2, preferred_element_type=jnp.float32))


def _build_table(hour_table, weekday_table, month_table, season_table, w, b):
    return pl.pallas_call(
        _table_body,
        out_shape=jax.ShapeDtypeStruct((N_ROWS, D), jnp.float32),
    )(hour_table, weekday_table, month_table, season_table, w, b)


def _idx_body(ts_ref, idx_ref):
    t = ts_ref[...]
    h = (t // 60) % 24
    wd = (t // 1440) % 7
    m = (t // 43200) % 12
    idx_ref[...] = h * 84 + wd * 12 + m


def _build_idx(timestamps):
    blk = 512
    return pl.pallas_call(
        _idx_body,
        grid=(B // blk,),
        in_specs=[pl.BlockSpec((blk, S), lambda i: (i, 0))],
        out_specs=pl.BlockSpec((blk, S), lambda i: (i, 0)),
        out_shape=jax.ShapeDtypeStruct((B, S), jnp.int32),
    )(timestamps)


def _sc_gather_body(ft_hbm, idxa_hbm, idxb_hbm, out_hbm,
                    idxa_all, idxb_all, rows_a0, rows_b0, rows_a1, rows_b1,
                    gsem0, gsem1, osem0, osem1):
    wid = lax.axis_index("s") * NC + lax.axis_index("c")
    pair0 = wid * PAIR_PER_W
    rows_a = (rows_a0, rows_a1)
    rows_b = (rows_b0, rows_b1)
    gsem = (gsem0, gsem1)
    osem = (osem0, osem1)

    pltpu.sync_copy(idxa_hbm.at[pl.ds(pair0, PAIR_PER_W)], idxa_all)
    pltpu.sync_copy(idxb_hbm.at[pl.ds(pair0, PAIR_PER_W)], idxb_all)

    def issue_gathers(i, k):
        off = i * HALF
        pltpu.async_copy(ft_hbm.at[idxa_all.at[pl.ds(off, HALF)]], rows_a[k], gsem[k])
        pltpu.async_copy(ft_hbm.at[idxb_all.at[pl.ds(off, HALF)]], rows_b[k], gsem[k])

    def wait_gathers(k):
        pltpu.make_async_copy(ft_hbm.at[pl.ds(0, HALF)], rows_a[k], gsem[k]).wait()
        pltpu.make_async_copy(ft_hbm.at[pl.ds(0, HALF)], rows_b[k], gsem[k]).wait()

    def issue_writes(i, k):
        base = pair0 + i * HALF
        pltpu.async_copy(rows_a[k], out_hbm.at[pl.ds(base, HALF), pl.ds(0, D)], osem[k])
        pltpu.async_copy(rows_b[k], out_hbm.at[pl.ds(base, HALF), pl.ds(D, D)], osem[k])

    def wait_writes(k):
        pltpu.make_async_copy(rows_a[k], out_hbm.at[pl.ds(0, HALF), pl.ds(0, D)], osem[k]).wait()
        pltpu.make_async_copy(rows_b[k], out_hbm.at[pl.ds(0, HALF), pl.ds(D, D)], osem[k]).wait()

    issue_gathers(0, 0)
    issue_gathers(1, 1)
    wait_gathers(0)
    issue_writes(0, 0)

    def steady(j, _):
        for k in (0, 1):
            i = 2 + 2 * j + k
            wait_writes(k)
            wait_gathers(1 - k)
            issue_writes(i - 1, 1 - k)
            issue_gathers(i, k)
        return 0

    lax.fori_loop(0, (N_CHUNKS - 2) // 2, steady, 0)

    wait_gathers(1)
    issue_writes(N_CHUNKS - 1, 1)
    wait_writes(0)
    wait_writes(1)


@functools.cache
def _sc_gather():
    return functools.partial(
        pl.kernel,
        mesh=plsc.VectorSubcoreMesh(core_axis_name="c", subcore_axis_name="s"),
        out_type=jax.ShapeDtypeStruct((N_PAIR, 2 * D), jnp.float32),
        scratch_types=[
            pltpu.VMEM((PAIR_PER_W,), jnp.int32),
            pltpu.VMEM((PAIR_PER_W,), jnp.int32),
            pltpu.VMEM((HALF, D), jnp.float32),
            pltpu.VMEM((HALF, D), jnp.float32),
            pltpu.VMEM((HALF, D), jnp.float32),
            pltpu.VMEM((HALF, D), jnp.float32),
            pltpu.SemaphoreType.DMA,
            pltpu.SemaphoreType.DMA,
            pltpu.SemaphoreType.DMA,
            pltpu.SemaphoreType.DMA,
        ],
        compiler_params=pltpu.CompilerParams(use_tc_tiling_on_sc=False),
    )(_sc_gather_body)


def _transpose_body(x_ref, o_ref):
    i = pl.program_id(0)
    x = x_ref[...]
    a = jnp.where(i < HB // BB, x[:, :, :D], x[:, :, D:])
    o_ref[...] = jnp.transpose(a, (0, 2, 1))


def _transpose(x3):
    return pl.pallas_call(
        _transpose_body,
        grid=(B // BB,),
        in_specs=[pl.BlockSpec((S, BB, 128), lambda i: (0, i % (HB // BB), 0))],
        out_specs=pl.BlockSpec((S, D, BB), lambda i: (0, 0, i)),
        out_shape=jax.ShapeDtypeStruct((S, D, B), jnp.float32),
    )(x3)


def kernel(timestamps, hour_table, weekday_table, month_table, season_table, W, b):
    ft = _build_table(hour_table, weekday_table, month_table, season_table, W, b)
    idx = _build_idx(timestamps)
    idx_a = idx[:HB, :].T.reshape(N_PAIR)
    idx_b = idx[HB:, :].T.reshape(N_PAIR)
    flat2 = _sc_gather()(ft, idx_a, idx_b)
    ot = _transpose(flat2.reshape(S, HB, 128))
    return jnp.transpose(ot, (2, 0, 1))
